# baseline (speedup 1.0000x reference)
import functools

import jax
import jax.numpy as jnp
from jax import lax
from jax.experimental import pallas as pl
from jax.experimental.pallas import tpu as pltpu, tpu_sc as plsc

_NUM_CAMS = 100000
_D = 9


@functools.cache
def _make_gather(B):
    info = plsc.get_sparse_core_info()
    NC, NS = info.num_cores, info.num_subcores
    NW = NC * NS
    b_per_w = B // NW
    assert B % NW == 0 and b_per_w % 8 == 0
    mesh = plsc.VectorSubcoreMesh(core_axis_name="c", subcore_axis_name="s")

    @functools.partial(
        pl.kernel,
        mesh=mesh,
        compiler_params=pltpu.CompilerParams(
            use_tc_tiling_on_sc=False, needs_layout_passes=False
        ),
        out_type=jax.ShapeDtypeStruct((B * _D,), jnp.float32),
        scratch_types=[
            pltpu.VMEM((b_per_w,), jnp.int32),
            pltpu.VMEM((b_per_w * _D,), jnp.float32),
            [pltpu.SemaphoreType.DMA for _ in range(_D)],
            pltpu.SemaphoreType.DMA,
        ],
    )
    def k(idx_hbm, table_hbm, out_hbm, idx_v, rows_v, sems, out_sem):
        wid = lax.axis_index("s") * NC + lax.axis_index("c")
        pltpu.sync_copy(idx_hbm.at[pl.ds(wid * b_per_w, b_per_w)], idx_v)
        gathers = [
            pltpu.async_copy(
                table_hbm.at[pl.ds(kk * _NUM_CAMS, _NUM_CAMS)].at[idx_v],
                rows_v.at[pl.ds(kk * b_per_w, b_per_w)],
                sems[kk],
            )
            for kk in range(_D)
        ]
        outs = []
        for kk in range(_D):
            gathers[kk].wait()
            outs.append(
                pltpu.async_copy(
                    rows_v.at[pl.ds(kk * b_per_w, b_per_w)],
                    out_hbm.at[pl.ds(kk * B + wid * b_per_w, b_per_w)],
                    out_sem,
                )
            )
        for o in outs:
            o.wait()

    return k


def kernel(i, param):
    B = i.shape[0]
    table = param.transpose(1, 2, 0).reshape(-1)
    out = _make_gather(B)(i.astype(jnp.int32), table)
    return out.reshape(3, 3, B).transpose(2, 0, 1)


# speedup vs baseline: 1.0070x; 1.0070x over previous
import functools

import jax
import jax.numpy as jnp
from jax import lax
from jax.experimental import pallas as pl
from jax.experimental.pallas import tpu as pltpu, tpu_sc as plsc

_NUM_CAMS = 100000
_D = 9


@functools.cache
def _make_gather(B):
    info = plsc.get_sparse_core_info()
    NC, NS = info.num_cores, info.num_subcores
    NW = NC * NS
    b_per_w = B // NW
    assert B % NW == 0 and b_per_w % 8 == 0
    mesh = plsc.VectorSubcoreMesh(core_axis_name="c", subcore_axis_name="s")

    @functools.partial(
        pl.kernel,
        mesh=mesh,
        compiler_params=pltpu.CompilerParams(
            use_tc_tiling_on_sc=False,
            needs_layout_passes=False,
            allow_input_fusion=[False, True],
        ),
        out_type=jax.ShapeDtypeStruct((B * _D,), jnp.float32),
        scratch_types=[
            pltpu.VMEM((b_per_w,), jnp.int32),
            pltpu.VMEM((b_per_w * _D,), jnp.float32),
            [pltpu.SemaphoreType.DMA for _ in range(_D)],
            pltpu.SemaphoreType.DMA,
        ],
    )
    def k(idx_hbm, table_hbm, out_hbm, idx_v, rows_v, sems, out_sem):
        wid = lax.axis_index("s") * NC + lax.axis_index("c")
        pltpu.sync_copy(idx_hbm.at[pl.ds(wid * b_per_w, b_per_w)], idx_v)
        gathers = [
            pltpu.async_copy(
                table_hbm.at[pl.ds(kk * _NUM_CAMS, _NUM_CAMS)].at[idx_v],
                rows_v.at[pl.ds(kk * b_per_w, b_per_w)],
                sems[kk],
            )
            for kk in range(_D)
        ]
        outs = []
        for kk in range(_D):
            gathers[kk].wait()
            outs.append(
                pltpu.async_copy(
                    rows_v.at[pl.ds(kk * b_per_w, b_per_w)],
                    out_hbm.at[pl.ds(kk * B + wid * b_per_w, b_per_w)],
                    out_sem,
                )
            )
        for o in outs:
            o.wait()

    return k


def kernel(i, param):
    B = i.shape[0]
    table = param.transpose(1, 2, 0).reshape(-1)
    out = _make_gather(B)(i.astype(jnp.int32), table)
    return out.reshape(3, 3, B).transpose(2, 0, 1)


# speedup vs baseline: 1.0083x; 1.0014x over previous
import functools

import jax
import jax.numpy as jnp
from jax import lax
from jax.experimental import pallas as pl
from jax.experimental.pallas import tpu as pltpu, tpu_sc as plsc

_NUM_CAMS = 100000
_D = 9


@functools.cache
def _make_gather(B):
    info = plsc.get_sparse_core_info()
    NC, NS = info.num_cores, info.num_subcores
    NW = NC * NS
    b_per_w = B // NW
    assert B % NW == 0 and b_per_w % 8 == 0
    mesh = plsc.VectorSubcoreMesh(core_axis_name="c", subcore_axis_name="s")

    @functools.partial(
        pl.kernel,
        mesh=mesh,
        compiler_params=pltpu.CompilerParams(
            use_tc_tiling_on_sc=False, needs_layout_passes=False
        ),
        out_type=jax.ShapeDtypeStruct((B * _D,), jnp.float32),
        scratch_types=[
            pltpu.VMEM((b_per_w,), jnp.int32),
            pltpu.VMEM((b_per_w * _D,), jnp.float32),
            [pltpu.SemaphoreType.DMA for _ in range(_D)],
            pltpu.SemaphoreType.DMA,
        ],
    )
    def k(idx_hbm, table_hbm, out_hbm, idx_v, rows_v, sems, out_sem):
        wid = lax.axis_index("s") * NC + lax.axis_index("c")
        pltpu.sync_copy(idx_hbm.at[pl.ds(wid * b_per_w, b_per_w)], idx_v)
        gathers = [
            pltpu.async_copy(
                table_hbm.at[pl.ds(kk * _NUM_CAMS, _NUM_CAMS)].at[idx_v],
                rows_v.at[pl.ds(kk * b_per_w, b_per_w)],
                sems[kk],
            )
            for kk in range(_D)
        ]
        outs = []
        for kk in range(_D):
            gathers[kk].wait()
            outs.append(
                pltpu.async_copy(
                    rows_v.at[pl.ds(kk * b_per_w, b_per_w)],
                    out_hbm.at[pl.ds(kk * B + wid * b_per_w, b_per_w)],
                    out_sem,
                )
            )
        for o in outs:
            o.wait()

    return k


def kernel(i, param):
    B = i.shape[0]
    table = param.transpose(1, 2, 0).reshape(-1)
    out = _make_gather(B)(i.astype(jnp.int32), table)
    return out.reshape(3, 3, B).transpose(2, 0, 1)
